# baseline (speedup 1.0000x reference)
import jax
import jax.numpy as jnp
from jax.experimental import pallas as pl
from jax.experimental.pallas import tpu as pltpu

N = 32768
E = 128
A = 8
B = 16
BLK = 8192
HLF = BLK // 2
NB = N // BLK
NEG = -1e30


def _body(bi_ref, x1_ref, x2_ref, wd_ref, bd_ref, wa_ref, ba_ref, out_ref,
          part_s, m_s, s_s):
    e = pl.program_id(0)
    i = pl.program_id(1)

    @pl.when(jnp.logical_and(e == 0, i == 0))
    def _init():
        m_s[...] = jnp.full((B, 1), NEG, jnp.float32)
        s_s[...] = jnp.zeros((B, 1), jnp.float32)

    @pl.when(e == 0)
    def _epoch0():
        dn = (((0,), (1,)), ((), ()))
        x1 = x1_ref[...]
        x2 = x2_ref[...]
        devT = jnp.concatenate(
            [jax.lax.dot_general(wd_ref[...], x1, dn,
                                 preferred_element_type=jnp.float32),
             jax.lax.dot_general(wd_ref[...], x2, dn,
                                 preferred_element_type=jnp.float32)],
            axis=1) + bd_ref[0, 0]
        actT = jnp.concatenate(
            [jax.lax.dot_general(wa_ref[...], x1, dn,
                                 preferred_element_type=jnp.float32),
             jax.lax.dot_general(wa_ref[...], x2, dn,
                                 preferred_element_type=jnp.float32)],
            axis=1) + ba_ref[...]
        rmax = jnp.max(actT, axis=0, keepdims=True)
        sh = actT - rmax
        lse = jnp.log(jnp.sum(jnp.exp(sh), axis=0, keepdims=True))
        part_s[:, pl.ds(i * BLK, BLK)] = sh - lse + devT

        biT = bi_ref[...]
        seg = jax.lax.broadcasted_iota(jnp.int32, (B, 1), 0)
        mask = biT == seg
        devb = jnp.where(mask, devT, NEG)
        bmax = jnp.max(devb, axis=1, keepdims=True)
        m_old = m_s[...]
        m_new = jnp.maximum(m_old, bmax)
        ssum = jnp.sum(jnp.exp(devb - m_new), axis=1, keepdims=True)
        s_s[...] = s_s[...] * jnp.exp(m_old - m_new) + ssum
        m_s[...] = m_new

    @pl.when(e == 1)
    def _epoch1():
        c = m_s[...] + jnp.log(s_s[...])
        biT = bi_ref[...]
        seg = jax.lax.broadcasted_iota(jnp.int32, (B, 1), 0)
        mask = biT == seg
        corr = jnp.sum(jnp.where(mask, c, 0.0), axis=0, keepdims=True)
        outT = part_s[:, pl.ds(i * BLK, BLK)] - corr
        out_ref[...] = outT.T


def kernel(embed_states, batch_index, W_dev, b_dev, W_act, b_act):
    bi = batch_index.astype(jnp.int32).reshape(1, N)
    bd = b_dev.reshape(1, 1)
    ba = b_act.reshape(A, 1)

    grid = (2, NB)
    out = pl.pallas_call(
        _body,
        grid=grid,
        in_specs=[
            pl.BlockSpec((1, BLK), lambda e, i: (0, i)),
            pl.BlockSpec((HLF, E), lambda e, i: (2 * i * (1 - e), 0)),
            pl.BlockSpec((HLF, E), lambda e, i: ((2 * i + 1) * (1 - e), 0)),
            pl.BlockSpec((E, 1), lambda e, i: (0, 0)),
            pl.BlockSpec((1, 1), lambda e, i: (0, 0)),
            pl.BlockSpec((E, A), lambda e, i: (0, 0)),
            pl.BlockSpec((A, 1), lambda e, i: (0, 0)),
        ],
        out_specs=pl.BlockSpec((BLK, A), lambda e, i: (e * i, 0)),
        out_shape=jax.ShapeDtypeStruct((N, A), jnp.float32),
        scratch_shapes=[
            pltpu.VMEM((A, N), jnp.float32),
            pltpu.VMEM((B, 1), jnp.float32),
            pltpu.VMEM((B, 1), jnp.float32),
        ],
        compiler_params=pltpu.CompilerParams(
            dimension_semantics=("arbitrary", "arbitrary"),
        ),
    )(bi, embed_states, embed_states, W_dev, bd, W_act, ba)
    return out


# speedup vs baseline: 1.1220x; 1.1220x over previous
import jax
import jax.numpy as jnp
from jax.experimental import pallas as pl
from jax.experimental.pallas import tpu as pltpu

N = 32768
E = 128
A = 8
B = 16
BLK = 8192
NB = N // BLK
NEG = -1e30


def _body(bi_ref, x_ref, w_ref, bb_ref, out_ref, part_s, m_s, s_s):
    e = pl.program_id(0)
    i = pl.program_id(1)

    @pl.when(jnp.logical_and(e == 0, i == 0))
    def _init():
        m_s[...] = jnp.full((B, 1), NEG, jnp.float32)
        s_s[...] = jnp.zeros((B, 1), jnp.float32)

    @pl.when(e == 0)
    def _epoch0():
        x = x_ref[...]
        dn = (((0,), (1,)), ((), ()))
        zT = jax.lax.dot_general(
            w_ref[...], x, dn, preferred_element_type=jnp.float32
        ) + bb_ref[...]
        actT = zT[:A]
        devT = zT[A:A + 1]
        rmax = jnp.max(actT, axis=0, keepdims=True)
        sh = actT - rmax
        lse = jnp.log(jnp.sum(jnp.exp(sh), axis=0, keepdims=True))
        part_s[:, pl.ds(i * BLK, BLK)] = sh - lse + devT

        biT = bi_ref[...]
        seg = jax.lax.broadcasted_iota(jnp.int32, (B, 1), 0)
        mask = biT == seg
        devb = jnp.where(mask, devT, NEG)
        bmax = jnp.max(devb, axis=1, keepdims=True)
        m_old = m_s[...]
        m_new = jnp.maximum(m_old, bmax)
        ssum = jnp.sum(jnp.exp(devb - m_new), axis=1, keepdims=True)
        s_s[...] = s_s[...] * jnp.exp(m_old - m_new) + ssum
        m_s[...] = m_new

    @pl.when(e == 1)
    def _epoch1():
        c = m_s[...] + jnp.log(s_s[...])
        biT = bi_ref[...]
        seg = jax.lax.broadcasted_iota(jnp.int32, (B, 1), 0)
        mask = biT == seg
        corr = jnp.sum(jnp.where(mask, c, 0.0), axis=0, keepdims=True)
        outT = part_s[:, pl.ds(i * BLK, BLK)] - corr
        out_ref[...] = outT.T


def kernel(embed_states, batch_index, W_dev, b_dev, W_act, b_act):
    bi = batch_index.astype(jnp.int32).reshape(1, N)
    w = jnp.concatenate([W_act, W_dev], axis=1)
    bb = jnp.concatenate([b_act, b_dev]).reshape(A + 1, 1)

    grid = (2, NB)
    out = pl.pallas_call(
        _body,
        grid=grid,
        in_specs=[
            pl.BlockSpec((1, BLK), lambda e, i: (0, i)),
            pl.BlockSpec((BLK, E), lambda e, i: (i * (1 - e) + (NB - 1) * e, 0)),
            pl.BlockSpec((E, A + 1), lambda e, i: (0, 0)),
            pl.BlockSpec((A + 1, 1), lambda e, i: (0, 0)),
        ],
        out_specs=pl.BlockSpec((BLK, A), lambda e, i: (e * i, 0)),
        out_shape=jax.ShapeDtypeStruct((N, A), jnp.float32),
        scratch_shapes=[
            pltpu.VMEM((A, N), jnp.float32),
            pltpu.VMEM((B, 1), jnp.float32),
            pltpu.VMEM((B, 1), jnp.float32),
        ],
        compiler_params=pltpu.CompilerParams(
            dimension_semantics=("arbitrary", "arbitrary"),
        ),
    )(bi, embed_states, w, bb)
    return out
